# baseline (speedup 1.0000x reference)
import functools

import jax
import jax.numpy as jnp
import numpy as np
from jax import lax
from jax.experimental import pallas as pl
from jax.experimental.pallas import tpu as pltpu
from jax.experimental.pallas import tpu_sc as plsc

NATOMS = 10000
NEDGES = 320000
EMB = 128
NRBF = 16
NHID = 2
INV_SQRT_2 = 1.0 / np.sqrt(2.0)

EDGE_BLK = 2560
NTILE = 16
GROWS = 2
GEDGES = GROWS * 128
NFULL = NEDGES // GEDGES
G_PER = NFULL // NTILE
HGRP = G_PER // 2
CATOMS = NATOMS // 2
NDUMP = 8
ACC_ROWS = CATOMS + NDUMP
AINIT = 312


def _swish(x):
    return x * jax.nn.sigmoid(x)


def _bdot(a, b):
    return jnp.dot(a.astype(jnp.bfloat16), b.astype(jnp.bfloat16),
                   preferred_element_type=jnp.float32)


def _swish16(x):
    xb = x.astype(jnp.bfloat16)
    return xb * jax.nn.sigmoid(xb)


def _payload_body(m_ref, rbft_ref, wrbf_ref, x_ref):
    rbfm = lax.dot_general(
        rbft_ref[...].astype(jnp.bfloat16), wrbf_ref[...].astype(jnp.bfloat16),
        (((0,), (0,)), ((), ())), preferred_element_type=jnp.float32)
    x_ref[...] = m_ref[...] * rbfm


def _payload_call(m, rbft, W_rbf):
    nblk = NEDGES // EDGE_BLK
    full = lambda shape: pl.BlockSpec(shape, lambda i: (0,) * len(shape))
    return pl.pallas_call(
        _payload_body,
        grid=(nblk,),
        in_specs=[
            pl.BlockSpec((EDGE_BLK, EMB), lambda i: (i, 0)),
            pl.BlockSpec((NRBF, EDGE_BLK), lambda i: (0, i)),
            full((NRBF, EMB)),
        ],
        out_specs=pl.BlockSpec((EDGE_BLK, EMB), lambda i: (i, 0)),
        out_shape=jax.ShapeDtypeStruct((NEDGES, EMB), jnp.float32),
    )(m, rbft, W_rbf)


def _force_body(m_ref, rbft_ref, wrbfF_ref, wf0_ref, wfres_ref,
                woutf_ref, xf_ref):
    m = m_ref[...]
    rbfmF = lax.dot_general(
        rbft_ref[...].astype(jnp.bfloat16),
        wrbfF_ref[...].astype(jnp.bfloat16),
        (((0,), (0,)), ((), ())), preferred_element_type=jnp.float32)
    f = _swish16(_bdot(m, wf0_ref[...]))
    for i in range(NHID):
        y = f
        for j in range(2):
            y = _swish16(_bdot(y, wfres_ref[i, j]))
        f = (f + y) * jnp.bfloat16(INV_SQRT_2)
    xf = f.astype(jnp.float32) * rbfmF
    xf_row = lax.dot_general(woutf_ref[...], xf, (((1,), (1,)), ((), ())),
                             preferred_element_type=jnp.float32)
    xf_ref[...] = xf_row.reshape(1, 1, EDGE_BLK)


def _force_call(m, rbft, W_rbf_F, W_f0, W_f_res, W_out_f_row):
    nblk = NEDGES // EDGE_BLK
    full = lambda shape: pl.BlockSpec(shape, lambda i: (0,) * len(shape))
    return pl.pallas_call(
        _force_body,
        grid=(nblk,),
        in_specs=[
            pl.BlockSpec((EDGE_BLK, EMB), lambda i: (i, 0)),
            pl.BlockSpec((NRBF, EDGE_BLK), lambda i: (0, i)),
            full((NRBF, EMB)),
            full((EMB, EMB)),
            full((NHID, 2, EMB, EMB)),
            full((1, EMB)),
        ],
        out_specs=pl.BlockSpec((1, 1, EDGE_BLK), lambda i: (i, 0, 0)),
        out_shape=jax.ShapeDtypeStruct((nblk, 1, EDGE_BLK), jnp.float32),
    )(m, rbft, W_rbf_F, W_f0, W_f_res, W_out_f_row)


def _atom_body(parts_ref, we0_ref, weres_ref, woute_ref, out_ref):
    x = jnp.concatenate([parts_ref[0], parts_ref[1]], axis=0)
    x = _swish(_bdot(x, we0_ref[...]))
    for i in range(NHID):
        y = x
        for j in range(2):
            y = _swish(_bdot(y, weres_ref[i, j]))
        x = (x + y) * INV_SQRT_2
    out_ref[...] = jnp.dot(x, woute_ref[...],
                           preferred_element_type=jnp.float32)


def _atom_call(parts, W_e0_s, W_e_res, W_out_e):
    return pl.pallas_call(
        _atom_body,
        out_shape=jax.ShapeDtypeStruct((NATOMS, 1), jnp.float32),
    )(parts, W_e0_s, W_e_res, W_out_e)


def _sc_scatter_body(x_hbm, idx_hbm, zeros_hbm, out_hbm,
                     idx_v0, idx_v1, x_v0, x_v1, acc, sem0, sem1):
    c = lax.axis_index("c")
    s = lax.axis_index("s")

    ab = s * AINIT
    pltpu.sync_copy(zeros_hbm.at[pl.ds(ab, AINIT)],
                    acc.at[pl.ds(ab, AINIT)])

    @pl.when(s == 0)
    def _():
        pltpu.sync_copy(zeros_hbm.at[pl.ds(16 * AINIT, ACC_ROWS - 16 * AINIT)],
                        acc.at[pl.ds(16 * AINIT, ACC_ROWS - 16 * AINIT)])

    plsc.subcore_barrier()

    base = c * CATOMS
    dump = CATOMS + jnp.bitwise_and(lax.iota(jnp.int32, 16), NDUMP - 1)

    def start(idx_v, x_v, sem, g):
        pltpu.async_copy(idx_hbm.at[g], idx_v, sem)
        pltpu.async_copy(x_hbm.at[pl.ds(g * GEDGES, GEDGES)], x_v, sem)

    def drain(idx_v, x_v, sem):
        pltpu.make_async_copy(idx_hbm.at[0], idx_v, sem).wait()
        pltpu.make_async_copy(x_hbm.at[pl.ds(0, GEDGES)], x_v, sem).wait()

    def xform_scatter(idx_v, x_v, nrows):
        for k in range(nrows):
            for q in range(128 // 16):
                t = idx_v[k, pl.ds(q * 16, 16)] - base
                ok = (t >= 0) & (t < CATOMS)
                idx_v[k, pl.ds(q * 16, 16)] = jnp.where(ok, t, dump)
        for k in range(nrows):
            pltpu.sync_copy(x_v.at[pl.ds(k * 128, 128)],
                            acc.at[idx_v.at[k]], add=True)

    g0 = s * G_PER
    start(idx_v0, x_v0, sem0, g0)

    def body(j, carry):
        g = g0 + 2 * j
        drain(idx_v0, x_v0, sem0)
        start(idx_v1, x_v1, sem1, g + 1)
        xform_scatter(idx_v0, x_v0, GROWS)
        drain(idx_v1, x_v1, sem1)
        start(idx_v0, x_v0, sem0, g + 2)
        xform_scatter(idx_v1, x_v1, GROWS)
        return carry

    lax.fori_loop(0, HGRP, body, 0)
    drain(idx_v0, x_v0, sem0)

    @pl.when(s < NFULL - NTILE * G_PER)
    def _():
        g = NTILE * G_PER + s
        pltpu.sync_copy(idx_hbm.at[g], idx_v0)
        pltpu.sync_copy(x_hbm.at[pl.ds(g * GEDGES, GEDGES)], x_v0)
        xform_scatter(idx_v0, x_v0, GROWS)

    plsc.subcore_barrier()
    pltpu.sync_copy(acc.at[pl.ds(ab, AINIT)],
                    out_hbm.at[c, pl.ds(ab, AINIT)])

    @pl.when(s == 0)
    def _():
        pltpu.sync_copy(acc.at[pl.ds(16 * AINIT, CATOMS - 16 * AINIT)],
                        out_hbm.at[c, pl.ds(16 * AINIT, CATOMS - 16 * AINIT)])


def _sc_scatter(x, idx3):
    mesh = plsc.VectorSubcoreMesh(core_axis_name="c", subcore_axis_name="s")
    zeros = jnp.zeros((ACC_ROWS, EMB), jnp.float32)
    fn = functools.partial(
        pl.kernel,
        mesh=mesh,
        out_type=jax.ShapeDtypeStruct((2, CATOMS, EMB), jnp.float32),
        scratch_types=[
            pltpu.VMEM((GROWS, 128), jnp.int32),
            pltpu.VMEM((GROWS, 128), jnp.int32),
            pltpu.VMEM((GEDGES, EMB), jnp.float32),
            pltpu.VMEM((GEDGES, EMB), jnp.float32),
            pltpu.VMEM_SHARED((ACC_ROWS, EMB), jnp.float32),
            pltpu.SemaphoreType.DMA,
            pltpu.SemaphoreType.DMA,
        ],
    )(_sc_scatter_body)
    return fn(x, idx3, zeros)


def kernel(h, m, rbf, id_j, W_rbf, scale_sum, W_e0, W_e_res, W_out_e,
           W_f0, W_f_res, W_rbf_F, scale_rbf, W_out_f):
    del h
    rbft = rbf.T
    x = _payload_call(m, rbft, W_rbf)
    x_F3 = _force_call(m, rbft, W_rbf_F, W_f0, W_f_res,
                       (W_out_f * scale_rbf).reshape(1, EMB))
    x_F = x_F3.reshape(NEDGES, 1)
    idx3 = id_j.reshape(NFULL, GROWS, 128)
    parts = _sc_scatter(x, idx3)
    x_E = _atom_call(parts, W_e0 * scale_sum, W_e_res, W_out_e)
    return (x_E, x_F)


# speedup vs baseline: 1.1515x; 1.1515x over previous
import functools

import jax
import jax.numpy as jnp
import numpy as np
from jax import lax
from jax.experimental import pallas as pl
from jax.experimental.pallas import tpu as pltpu
from jax.experimental.pallas import tpu_sc as plsc

NATOMS = 10000
NEDGES = 320000
EMB = 128
NRBF = 16
NHID = 2
INV_SQRT_2 = 1.0 / np.sqrt(2.0)

EDGE_BLK = 6400
NTILE = 16
GROWS = 2
GEDGES = GROWS * 128
NFULL = NEDGES // GEDGES
G_PER = NFULL // NTILE
HGRP = G_PER // 2
CATOMS = NATOMS // 2
NDUMP = 8
ACC_ROWS = CATOMS + NDUMP
AINIT = 312


def _swish(x):
    return x * jax.nn.sigmoid(x)


def _bdot(a, b):
    return jnp.dot(a.astype(jnp.bfloat16), b.astype(jnp.bfloat16),
                   preferred_element_type=jnp.float32)


def _swish16(x):
    xb = x.astype(jnp.bfloat16)
    return xb * jax.nn.sigmoid(xb)


def _payload_body(m_ref, rbft_ref, wrbf_ref, x_ref):
    rbfm = lax.dot_general(
        rbft_ref[...].astype(jnp.bfloat16), wrbf_ref[...].astype(jnp.bfloat16),
        (((0,), (0,)), ((), ())), preferred_element_type=jnp.float32)
    x_ref[...] = m_ref[...] * rbfm


def _payload_call(m, rbft, W_rbf):
    nblk = NEDGES // EDGE_BLK
    full = lambda shape: pl.BlockSpec(shape, lambda i: (0,) * len(shape))
    return pl.pallas_call(
        _payload_body,
        grid=(nblk,),
        in_specs=[
            pl.BlockSpec((EDGE_BLK, EMB), lambda i: (i, 0)),
            pl.BlockSpec((NRBF, EDGE_BLK), lambda i: (0, i)),
            full((NRBF, EMB)),
        ],
        out_specs=pl.BlockSpec((EDGE_BLK, EMB), lambda i: (i, 0)),
        out_shape=jax.ShapeDtypeStruct((NEDGES, EMB), jnp.float32),
    )(m, rbft, W_rbf)


def _force_body(m_ref, rbft_ref, wrbfF_ref, wf0_ref, wfres_ref,
                woutf_ref, xf_ref):
    m = m_ref[...]
    rbfmF = lax.dot_general(
        rbft_ref[...].astype(jnp.bfloat16),
        wrbfF_ref[...].astype(jnp.bfloat16),
        (((0,), (0,)), ((), ())), preferred_element_type=jnp.float32)
    f = _swish16(_bdot(m, wf0_ref[...]))
    for i in range(NHID):
        y = f
        for j in range(2):
            y = _swish16(_bdot(y, wfres_ref[i, j]))
        f = (f + y) * jnp.bfloat16(INV_SQRT_2)
    xf = f.astype(jnp.float32) * rbfmF
    xf_row = lax.dot_general(woutf_ref[...], xf, (((1,), (1,)), ((), ())),
                             preferred_element_type=jnp.float32)
    xf_ref[...] = xf_row.reshape(1, 1, EDGE_BLK)


def _force_call(m, rbft, W_rbf_F, W_f0, W_f_res, W_out_f_row):
    nblk = NEDGES // EDGE_BLK
    full = lambda shape: pl.BlockSpec(shape, lambda i: (0,) * len(shape))
    return pl.pallas_call(
        _force_body,
        grid=(nblk,),
        in_specs=[
            pl.BlockSpec((EDGE_BLK, EMB), lambda i: (i, 0)),
            pl.BlockSpec((NRBF, EDGE_BLK), lambda i: (0, i)),
            full((NRBF, EMB)),
            full((EMB, EMB)),
            full((NHID, 2, EMB, EMB)),
            full((1, EMB)),
        ],
        out_specs=pl.BlockSpec((1, 1, EDGE_BLK), lambda i: (i, 0, 0)),
        out_shape=jax.ShapeDtypeStruct((nblk, 1, EDGE_BLK), jnp.float32),
    )(m, rbft, W_rbf_F, W_f0, W_f_res, W_out_f_row)


def _atom_body(parts_ref, we0_ref, weres_ref, woute_ref, out_ref):
    x = jnp.concatenate([parts_ref[0], parts_ref[1]], axis=0)
    x = _swish(_bdot(x, we0_ref[...]))
    for i in range(NHID):
        y = x
        for j in range(2):
            y = _swish(_bdot(y, weres_ref[i, j]))
        x = (x + y) * INV_SQRT_2
    out_ref[...] = jnp.dot(x, woute_ref[...],
                           preferred_element_type=jnp.float32)


def _atom_call(parts, W_e0_s, W_e_res, W_out_e):
    return pl.pallas_call(
        _atom_body,
        out_shape=jax.ShapeDtypeStruct((NATOMS, 1), jnp.float32),
    )(parts, W_e0_s, W_e_res, W_out_e)


def _sc_scatter_body(x_hbm, idx_hbm, zeros_hbm, out_hbm,
                     idx_v0, idx_v1, x_v0, x_v1, acc, sem0, sem1):
    c = lax.axis_index("c")
    s = lax.axis_index("s")

    ab = s * AINIT
    pltpu.sync_copy(zeros_hbm.at[pl.ds(ab, AINIT)],
                    acc.at[pl.ds(ab, AINIT)])

    @pl.when(s == 0)
    def _():
        pltpu.sync_copy(zeros_hbm.at[pl.ds(16 * AINIT, ACC_ROWS - 16 * AINIT)],
                        acc.at[pl.ds(16 * AINIT, ACC_ROWS - 16 * AINIT)])

    plsc.subcore_barrier()

    base = c * CATOMS
    dump = CATOMS + jnp.bitwise_and(lax.iota(jnp.int32, 16), NDUMP - 1)

    def start(idx_v, x_v, sem, g):
        pltpu.async_copy(idx_hbm.at[g], idx_v, sem)
        pltpu.async_copy(x_hbm.at[pl.ds(g * GEDGES, GEDGES)], x_v, sem)

    def drain(idx_v, x_v, sem):
        pltpu.make_async_copy(idx_hbm.at[0], idx_v, sem).wait()
        pltpu.make_async_copy(x_hbm.at[pl.ds(0, GEDGES)], x_v, sem).wait()

    def xform_scatter(idx_v, x_v, nrows):
        for k in range(nrows):
            for q in range(128 // 16):
                t = idx_v[k, pl.ds(q * 16, 16)] - base
                ok = (t >= 0) & (t < CATOMS)
                idx_v[k, pl.ds(q * 16, 16)] = jnp.where(ok, t, dump)
        for k in range(nrows):
            pltpu.sync_copy(x_v.at[pl.ds(k * 128, 128)],
                            acc.at[idx_v.at[k]], add=True)

    g0 = s * G_PER
    start(idx_v0, x_v0, sem0, g0)

    def body(j, carry):
        g = g0 + 2 * j
        drain(idx_v0, x_v0, sem0)
        start(idx_v1, x_v1, sem1, g + 1)
        xform_scatter(idx_v0, x_v0, GROWS)
        drain(idx_v1, x_v1, sem1)
        start(idx_v0, x_v0, sem0, g + 2)
        xform_scatter(idx_v1, x_v1, GROWS)
        return carry

    lax.fori_loop(0, HGRP, body, 0)
    drain(idx_v0, x_v0, sem0)

    @pl.when(s < NFULL - NTILE * G_PER)
    def _():
        g = NTILE * G_PER + s
        pltpu.sync_copy(idx_hbm.at[g], idx_v0)
        pltpu.sync_copy(x_hbm.at[pl.ds(g * GEDGES, GEDGES)], x_v0)
        xform_scatter(idx_v0, x_v0, GROWS)

    plsc.subcore_barrier()
    pltpu.sync_copy(acc.at[pl.ds(ab, AINIT)],
                    out_hbm.at[c, pl.ds(ab, AINIT)])

    @pl.when(s == 0)
    def _():
        pltpu.sync_copy(acc.at[pl.ds(16 * AINIT, CATOMS - 16 * AINIT)],
                        out_hbm.at[c, pl.ds(16 * AINIT, CATOMS - 16 * AINIT)])


def _sc_scatter(x, idx3):
    mesh = plsc.VectorSubcoreMesh(core_axis_name="c", subcore_axis_name="s")
    zeros = jnp.zeros((ACC_ROWS, EMB), jnp.float32)
    fn = functools.partial(
        pl.kernel,
        mesh=mesh,
        out_type=jax.ShapeDtypeStruct((2, CATOMS, EMB), jnp.float32),
        scratch_types=[
            pltpu.VMEM((GROWS, 128), jnp.int32),
            pltpu.VMEM((GROWS, 128), jnp.int32),
            pltpu.VMEM((GEDGES, EMB), jnp.float32),
            pltpu.VMEM((GEDGES, EMB), jnp.float32),
            pltpu.VMEM_SHARED((ACC_ROWS, EMB), jnp.float32),
            pltpu.SemaphoreType.DMA,
            pltpu.SemaphoreType.DMA,
        ],
    )(_sc_scatter_body)
    return fn(x, idx3, zeros)


def kernel(h, m, rbf, id_j, W_rbf, scale_sum, W_e0, W_e_res, W_out_e,
           W_f0, W_f_res, W_rbf_F, scale_rbf, W_out_f):
    del h
    rbft = rbf.T
    x = _payload_call(m, rbft, W_rbf)
    x_F3 = _force_call(m, rbft, W_rbf_F, W_f0, W_f_res,
                       (W_out_f * scale_rbf).reshape(1, EMB))
    x_F = x_F3.reshape(NEDGES, 1)
    idx3 = id_j.reshape(NFULL, GROWS, 128)
    parts = _sc_scatter(x, idx3)
    x_E = _atom_call(parts, W_e0 * scale_sum, W_e_res, W_out_e)
    return (x_E, x_F)


# speedup vs baseline: 1.1803x; 1.0250x over previous
import functools

import jax
import jax.numpy as jnp
import numpy as np
from jax import lax
from jax.experimental import pallas as pl
from jax.experimental.pallas import tpu as pltpu
from jax.experimental.pallas import tpu_sc as plsc

NATOMS = 10000
NEDGES = 320000
EMB = 128
NRBF = 16
NHID = 2
INV_SQRT_2 = 1.0 / np.sqrt(2.0)

EDGE_BLK = 12800
NTILE = 16
GROWS = 2
GEDGES = GROWS * 128
NFULL = NEDGES // GEDGES
G_PER = NFULL // NTILE
HGRP = G_PER // 2
CATOMS = NATOMS // 2
NDUMP = 8
ACC_ROWS = CATOMS + NDUMP
AINIT = 312


def _swish(x):
    return x * jax.nn.sigmoid(x)


def _bdot(a, b):
    return jnp.dot(a.astype(jnp.bfloat16), b.astype(jnp.bfloat16),
                   preferred_element_type=jnp.float32)


def _swish16(x):
    xb = x.astype(jnp.bfloat16)
    return xb * jax.nn.sigmoid(xb)


def _payload_body(m_ref, rbft_ref, wrbf_ref, x_ref):
    rbfm = lax.dot_general(
        rbft_ref[...].astype(jnp.bfloat16), wrbf_ref[...].astype(jnp.bfloat16),
        (((0,), (0,)), ((), ())), preferred_element_type=jnp.float32)
    x_ref[...] = m_ref[...] * rbfm


def _payload_call(m, rbft, W_rbf):
    nblk = NEDGES // EDGE_BLK
    full = lambda shape: pl.BlockSpec(shape, lambda i: (0,) * len(shape))
    return pl.pallas_call(
        _payload_body,
        grid=(nblk,),
        in_specs=[
            pl.BlockSpec((EDGE_BLK, EMB), lambda i: (i, 0)),
            pl.BlockSpec((NRBF, EDGE_BLK), lambda i: (0, i)),
            full((NRBF, EMB)),
        ],
        out_specs=pl.BlockSpec((EDGE_BLK, EMB), lambda i: (i, 0)),
        out_shape=jax.ShapeDtypeStruct((NEDGES, EMB), jnp.float32),
    )(m, rbft, W_rbf)


def _force_body(m_ref, rbft_ref, wrbfF_ref, wf0_ref, wfres_ref,
                woutf_ref, xf_ref):
    m = m_ref[...]
    rbfmF = lax.dot_general(
        rbft_ref[...].astype(jnp.bfloat16),
        wrbfF_ref[...].astype(jnp.bfloat16),
        (((0,), (0,)), ((), ())), preferred_element_type=jnp.float32)
    f = _swish16(_bdot(m, wf0_ref[...]))
    for i in range(NHID):
        y = f
        for j in range(2):
            y = _swish16(_bdot(y, wfres_ref[i, j]))
        f = (f + y) * jnp.bfloat16(INV_SQRT_2)
    xf = f.astype(jnp.float32) * rbfmF
    xf_row = lax.dot_general(woutf_ref[...], xf, (((1,), (1,)), ((), ())),
                             preferred_element_type=jnp.float32)
    xf_ref[...] = xf_row.reshape(1, 1, EDGE_BLK)


def _force_call(m, rbft, W_rbf_F, W_f0, W_f_res, W_out_f_row):
    nblk = NEDGES // EDGE_BLK
    full = lambda shape: pl.BlockSpec(shape, lambda i: (0,) * len(shape))
    return pl.pallas_call(
        _force_body,
        grid=(nblk,),
        in_specs=[
            pl.BlockSpec((EDGE_BLK, EMB), lambda i: (i, 0)),
            pl.BlockSpec((NRBF, EDGE_BLK), lambda i: (0, i)),
            full((NRBF, EMB)),
            full((EMB, EMB)),
            full((NHID, 2, EMB, EMB)),
            full((1, EMB)),
        ],
        out_specs=pl.BlockSpec((1, 1, EDGE_BLK), lambda i: (i, 0, 0)),
        out_shape=jax.ShapeDtypeStruct((nblk, 1, EDGE_BLK), jnp.float32),
    )(m, rbft, W_rbf_F, W_f0, W_f_res, W_out_f_row)


def _atom_body(parts_ref, we0_ref, weres_ref, woute_ref, out_ref):
    x = jnp.concatenate([parts_ref[0], parts_ref[1]], axis=0)
    x = _swish(_bdot(x, we0_ref[...]))
    for i in range(NHID):
        y = x
        for j in range(2):
            y = _swish(_bdot(y, weres_ref[i, j]))
        x = (x + y) * INV_SQRT_2
    out_ref[...] = jnp.dot(x, woute_ref[...],
                           preferred_element_type=jnp.float32)


def _atom_call(parts, W_e0_s, W_e_res, W_out_e):
    return pl.pallas_call(
        _atom_body,
        out_shape=jax.ShapeDtypeStruct((NATOMS, 1), jnp.float32),
    )(parts, W_e0_s, W_e_res, W_out_e)


def _sc_scatter_body(x_hbm, idx_hbm, zeros_hbm, out_hbm,
                     idx_v0, idx_v1, x_v0, x_v1, acc, sem0, sem1):
    c = lax.axis_index("c")
    s = lax.axis_index("s")

    ab = s * AINIT
    pltpu.sync_copy(zeros_hbm.at[pl.ds(ab, AINIT)],
                    acc.at[pl.ds(ab, AINIT)])

    @pl.when(s == 0)
    def _():
        pltpu.sync_copy(zeros_hbm.at[pl.ds(16 * AINIT, ACC_ROWS - 16 * AINIT)],
                        acc.at[pl.ds(16 * AINIT, ACC_ROWS - 16 * AINIT)])

    plsc.subcore_barrier()

    base = c * CATOMS
    dump = CATOMS + jnp.bitwise_and(lax.iota(jnp.int32, 16), NDUMP - 1)

    def start(idx_v, x_v, sem, g):
        pltpu.async_copy(idx_hbm.at[g], idx_v, sem)
        pltpu.async_copy(x_hbm.at[pl.ds(g * GEDGES, GEDGES)], x_v, sem)

    def drain(idx_v, x_v, sem):
        pltpu.make_async_copy(idx_hbm.at[0], idx_v, sem).wait()
        pltpu.make_async_copy(x_hbm.at[pl.ds(0, GEDGES)], x_v, sem).wait()

    def xform_scatter(idx_v, x_v, nrows):
        for k in range(nrows):
            for q in range(128 // 16):
                t = idx_v[k, pl.ds(q * 16, 16)] - base
                ok = (t >= 0) & (t < CATOMS)
                idx_v[k, pl.ds(q * 16, 16)] = jnp.where(ok, t, dump)
        for k in range(nrows):
            pltpu.sync_copy(x_v.at[pl.ds(k * 128, 128)],
                            acc.at[idx_v.at[k]], add=True)

    g0 = s * G_PER
    start(idx_v0, x_v0, sem0, g0)

    def body(j, carry):
        g = g0 + 2 * j
        drain(idx_v0, x_v0, sem0)
        start(idx_v1, x_v1, sem1, g + 1)
        xform_scatter(idx_v0, x_v0, GROWS)
        drain(idx_v1, x_v1, sem1)
        start(idx_v0, x_v0, sem0, g + 2)
        xform_scatter(idx_v1, x_v1, GROWS)
        return carry

    lax.fori_loop(0, HGRP, body, 0)
    drain(idx_v0, x_v0, sem0)

    @pl.when(s < NFULL - NTILE * G_PER)
    def _():
        g = NTILE * G_PER + s
        pltpu.sync_copy(idx_hbm.at[g], idx_v0)
        pltpu.sync_copy(x_hbm.at[pl.ds(g * GEDGES, GEDGES)], x_v0)
        xform_scatter(idx_v0, x_v0, GROWS)

    plsc.subcore_barrier()
    pltpu.sync_copy(acc.at[pl.ds(ab, AINIT)],
                    out_hbm.at[c, pl.ds(ab, AINIT)])

    @pl.when(s == 0)
    def _():
        pltpu.sync_copy(acc.at[pl.ds(16 * AINIT, CATOMS - 16 * AINIT)],
                        out_hbm.at[c, pl.ds(16 * AINIT, CATOMS - 16 * AINIT)])


def _sc_scatter(x, idx3):
    mesh = plsc.VectorSubcoreMesh(core_axis_name="c", subcore_axis_name="s")
    zeros = jnp.zeros((ACC_ROWS, EMB), jnp.float32)
    fn = functools.partial(
        pl.kernel,
        mesh=mesh,
        out_type=jax.ShapeDtypeStruct((2, CATOMS, EMB), jnp.float32),
        scratch_types=[
            pltpu.VMEM((GROWS, 128), jnp.int32),
            pltpu.VMEM((GROWS, 128), jnp.int32),
            pltpu.VMEM((GEDGES, EMB), jnp.float32),
            pltpu.VMEM((GEDGES, EMB), jnp.float32),
            pltpu.VMEM_SHARED((ACC_ROWS, EMB), jnp.float32),
            pltpu.SemaphoreType.DMA,
            pltpu.SemaphoreType.DMA,
        ],
    )(_sc_scatter_body)
    return fn(x, idx3, zeros)


def kernel(h, m, rbf, id_j, W_rbf, scale_sum, W_e0, W_e_res, W_out_e,
           W_f0, W_f_res, W_rbf_F, scale_rbf, W_out_f):
    del h
    rbft = rbf.T
    x = _payload_call(m, rbft, W_rbf)
    x_F3 = _force_call(m, rbft, W_rbf_F, W_f0, W_f_res,
                       (W_out_f * scale_rbf).reshape(1, EMB))
    x_F = x_F3.reshape(NEDGES, 1)
    idx3 = id_j.reshape(NFULL, GROWS, 128)
    parts = _sc_scatter(x, idx3)
    x_E = _atom_call(parts, W_e0 * scale_sum, W_e_res, W_out_e)
    return (x_E, x_F)
